# baseline (speedup 1.0000x reference)
import functools

import jax
import jax.numpy as jnp
from jax import lax
from jax.experimental import pallas as pl
from jax.experimental.pallas import tpu as pltpu
from jax.experimental.pallas import tpu_sc as plsc

N = 10000
HID = 64
NC = 2
NS = 16
L = 16
NW = NC * NS
CB = 128
N_ACC = 10240
RPT = N_ACC // NS


def _mesh():
    return plsc.VectorSubcoreMesh(core_axis_name="c", subcore_axis_name="s")


_SC_PARAMS = pltpu.CompilerParams(use_tc_tiling_on_sc=False)


def _fill(ref, rows, cols, val):
    @pl.loop(0, rows)
    def _(i):
        @pl.loop(0, cols, step=L)
        def _(j):
            ref.at[pl.ds(i, 1), pl.ds(j, L)][...] = jnp.full((1, L), val, jnp.float32)


def _sc_deg(dst3):
    chn = dst3.shape[1]

    @functools.partial(
        pl.kernel,
        out_type=jax.ShapeDtypeStruct((NC, N_ACC, L), jnp.float32),
        mesh=_mesh(),
        scratch_types=[
            pltpu.VMEM((chn, CB), jnp.int32),
            pltpu.VMEM((CB, L), jnp.float32),
            pltpu.VMEM((CB, L), jnp.float32),
            pltpu.VMEM_SHARED((N_ACC, L), jnp.float32),
            pltpu.SemaphoreType.DMA,
        ],
        compiler_params=_SC_PARAMS,
    )
    def k(dst_hbm, out_hbm, dst_v, ones_v, zb_v, acc_sh, sem):
        c = lax.axis_index("c")
        s = lax.axis_index("s")
        w = s * NC + c
        _fill(ones_v, CB, L, 1.0)
        _fill(zb_v, CB, L, 0.0)

        @pl.loop(0, RPT, step=CB)
        def _(r):
            pltpu.sync_copy(zb_v, acc_sh.at[pl.ds(s * RPT + r, CB)])

        plsc.subcore_barrier()
        pltpu.sync_copy(dst_hbm.at[w], dst_v)

        @pl.loop(0, chn, step=8)
        def _(j):
            for b in range(8):
                pltpu.async_copy(ones_v, acc_sh.at[dst_v.at[j + b]], sem, add=True)
            for b in range(8):
                pltpu.make_async_copy(ones_v, acc_sh.at[dst_v.at[j + b]], sem).wait()

        plsc.subcore_barrier()
        pltpu.sync_copy(acc_sh.at[pl.ds(s * RPT, RPT)],
                        out_hbm.at[c, pl.ds(s * RPT, RPT)])

    return k(dst3)


def _sc_agg(y, src3, dst3, d):
    chn = src3.shape[1]
    npt = N // NS

    @functools.partial(
        pl.kernel,
        out_type=jax.ShapeDtypeStruct((NC, N_ACC, d), jnp.float32),
        mesh=_mesh(),
        scratch_types=[
            pltpu.VMEM((chn, CB), jnp.int32),
            pltpu.VMEM((chn, CB), jnp.int32),
            pltpu.VMEM((CB, d), jnp.float32),
            pltpu.VMEM((CB, d), jnp.float32),
            pltpu.VMEM((CB, d), jnp.float32),
            pltpu.VMEM_SHARED((N, d), jnp.float32),
            pltpu.VMEM_SHARED((N_ACC, d), jnp.float32),
        ] + [pltpu.SemaphoreType.DMA] * 2,
        compiler_params=_SC_PARAMS,
    )
    def k(y_hbm, src_hbm, dst_hbm, out_hbm,
          src_v, dst_v, rows0, rows1, zb_v, y_sh, acc_sh, g0, g1):
        c = lax.axis_index("c")
        s = lax.axis_index("s")
        w = s * NC + c
        _fill(zb_v, CB, d, 0.0)

        @pl.loop(0, RPT, step=CB)
        def _(r):
            pltpu.sync_copy(zb_v, acc_sh.at[pl.ds(s * RPT + r, CB)])

        pltpu.sync_copy(y_hbm.at[pl.ds(s * npt, npt)], y_sh.at[pl.ds(s * npt, npt)])
        pltpu.sync_copy(src_hbm.at[w], src_v)
        pltpu.sync_copy(dst_hbm.at[w], dst_v)
        plsc.subcore_barrier()

        pltpu.async_copy(y_sh.at[src_v.at[0]], rows0, g0)

        @pl.loop(0, chn, step=2)
        def _(j):
            a1 = pltpu.async_copy(y_sh.at[src_v.at[j + 1]], rows1, g1)
            pltpu.make_async_copy(y_sh.at[src_v.at[j]], rows0, g0).wait()
            pltpu.sync_copy(rows0, acc_sh.at[dst_v.at[j]], add=True)

            @pl.when(j + 2 < chn)
            def _():
                pltpu.async_copy(y_sh.at[src_v.at[j + 2]], rows0, g0)

            a1.wait()
            pltpu.sync_copy(rows1, acc_sh.at[dst_v.at[j + 1]], add=True)

        plsc.subcore_barrier()
        pltpu.sync_copy(acc_sh.at[pl.ds(s * RPT, RPT)],
                        out_hbm.at[c, pl.ds(s * RPT, RPT)])

    return k(y, src3, dst3)


def _dinv2(degp):
    deg = degp[0, :N, 0:1] + degp[1, :N, 0:1] + 1.0
    return lax.rsqrt(deg)


def _tc1(x, W1, degp):
    def body(x_ref, w_ref, degp_ref, y_ref):
        dinv = _dinv2(degp_ref[...])
        xw = jnp.dot(x_ref[...], w_ref[...], preferred_element_type=jnp.float32)
        y_ref[...] = xw * dinv

    return pl.pallas_call(
        body, out_shape=jax.ShapeDtypeStruct((N, HID), jnp.float32),
    )(x, W1, degp)


def _tc2(a1p, y1, degp, b1, W2):
    def body(a1p_ref, y1_ref, degp_ref, b1_ref, w2_ref, y2p_ref):
        dinv = _dinv2(degp_ref[...])
        a1 = a1p_ref[0, :N, :] + a1p_ref[1, :N, :]
        h = jnp.maximum(dinv * (a1 + y1_ref[...]) + b1_ref[...], 0.0)
        hw = jnp.dot(h, w2_ref[...], preferred_element_type=jnp.float32)
        y2p_ref[...] = jnp.broadcast_to(dinv * hw, (N, L))

    return pl.pallas_call(
        body, out_shape=jax.ShapeDtypeStruct((N, L), jnp.float32),
    )(a1p, y1, degp, b1, W2)


def _tc3(a2p, y2p, degp, b2):
    def body(a2p_ref, y2p_ref, degp_ref, b2_ref, o_ref):
        dinv = _dinv2(degp_ref[...])
        a2 = a2p_ref[0, :N, 0:1] + a2p_ref[1, :N, 0:1]
        y2 = y2p_ref[:, 0:1]
        o_ref[...] = jax.nn.sigmoid(dinv * (a2 + y2) + b2_ref[...])

    return pl.pallas_call(
        body, out_shape=jax.ShapeDtypeStruct((N, 1), jnp.float32),
    )(a2p, y2p, degp, b2)


def kernel(x, edge_index, W1, b1, W2, b2):
    e = edge_index.shape[1]
    grain = NW * CB * 4
    chn = 4 * (-(-e // grain))
    e_pad = chn * NW * CB
    ei = edge_index.astype(jnp.int32)
    src = jnp.concatenate([ei[0], jnp.zeros((e_pad - e,), jnp.int32)])
    dst = jnp.concatenate([ei[1], jnp.full((e_pad - e,), N, jnp.int32)])
    src3 = src.reshape(NW, chn, CB)
    dst3 = dst.reshape(NW, chn, CB)

    degp = _sc_deg(dst3)
    y1 = _tc1(x, W1, degp)
    a1p = _sc_agg(y1, src3, dst3, HID)
    y2p = _tc2(a1p, y1, degp, b1, W2)
    a2p = _sc_agg(y2p, src3, dst3, L)
    return _tc3(a2p, y2p, degp, b2)


# speedup vs baseline: 1.0689x; 1.0689x over previous
import functools

import jax
import jax.numpy as jnp
from jax import lax
from jax.experimental import pallas as pl
from jax.experimental.pallas import tpu as pltpu
from jax.experimental.pallas import tpu_sc as plsc

N = 10000
HID = 64
NC = 2
NS = 16
L = 16
NW = NC * NS
CB = 128
N_ACC = 10240
RPT = N_ACC // NS


def _mesh():
    return plsc.VectorSubcoreMesh(core_axis_name="c", subcore_axis_name="s")


_SC_PARAMS = pltpu.CompilerParams(use_tc_tiling_on_sc=False)


def _fill(ref, rows, cols, val):
    @pl.loop(0, rows)
    def _(i):
        @pl.loop(0, cols, step=L)
        def _(j):
            ref.at[pl.ds(i, 1), pl.ds(j, L)][...] = jnp.full((1, L), val, jnp.float32)


def _sc_deg(ei3):
    chn = ei3.shape[2]

    @functools.partial(
        pl.kernel,
        out_type=jax.ShapeDtypeStruct((NC, N_ACC, L), jnp.float32),
        mesh=_mesh(),
        scratch_types=[
            pltpu.VMEM((chn, CB), jnp.int32),
            pltpu.VMEM((CB, L), jnp.float32),
            pltpu.VMEM((CB, L), jnp.float32),
            pltpu.VMEM_SHARED((N_ACC, L), jnp.float32),
            pltpu.SemaphoreType.DMA,
        ],
        compiler_params=_SC_PARAMS,
    )
    def k(ei_hbm, out_hbm, dst_v, ones_v, zb_v, acc_sh, sem):
        c = lax.axis_index("c")
        s = lax.axis_index("s")
        w = s * NC + c
        _fill(ones_v, CB, L, 1.0)
        _fill(zb_v, CB, L, 0.0)

        @pl.loop(0, RPT, step=CB)
        def _(r):
            pltpu.sync_copy(zb_v, acc_sh.at[pl.ds(s * RPT + r, CB)])

        plsc.subcore_barrier()
        pltpu.sync_copy(ei_hbm.at[1, w], dst_v)

        @pl.loop(0, chn, step=8)
        def _(j):
            for b in range(8):
                pltpu.async_copy(ones_v, acc_sh.at[dst_v.at[j + b]], sem, add=True)
            for b in range(8):
                pltpu.make_async_copy(ones_v, acc_sh.at[dst_v.at[j + b]], sem).wait()

        plsc.subcore_barrier()
        pltpu.sync_copy(acc_sh.at[pl.ds(s * RPT, RPT)],
                        out_hbm.at[c, pl.ds(s * RPT, RPT)])

    return k(ei3)


def _sc_agg(y, ei3, d):
    chn = ei3.shape[2]
    npt = N // NS

    @functools.partial(
        pl.kernel,
        out_type=jax.ShapeDtypeStruct((NC, N_ACC, d), jnp.float32),
        mesh=_mesh(),
        scratch_types=[
            pltpu.VMEM((chn, CB), jnp.int32),
            pltpu.VMEM((chn, CB), jnp.int32),
            pltpu.VMEM((CB, d), jnp.float32),
            pltpu.VMEM((CB, d), jnp.float32),
            pltpu.VMEM((CB, d), jnp.float32),
            pltpu.VMEM_SHARED((N, d), jnp.float32),
            pltpu.VMEM_SHARED((N_ACC, d), jnp.float32),
        ] + [pltpu.SemaphoreType.DMA] * 2,
        compiler_params=_SC_PARAMS,
    )
    def k(y_hbm, ei_hbm, out_hbm,
          src_v, dst_v, rows0, rows1, zb_v, y_sh, acc_sh, g0, g1):
        c = lax.axis_index("c")
        s = lax.axis_index("s")
        w = s * NC + c
        _fill(zb_v, CB, d, 0.0)

        @pl.loop(0, RPT, step=CB)
        def _(r):
            pltpu.sync_copy(zb_v, acc_sh.at[pl.ds(s * RPT + r, CB)])

        pltpu.sync_copy(y_hbm.at[pl.ds(s * npt, npt)], y_sh.at[pl.ds(s * npt, npt)])
        pltpu.sync_copy(ei_hbm.at[0, w], src_v)
        pltpu.sync_copy(ei_hbm.at[1, w], dst_v)
        plsc.subcore_barrier()

        pltpu.async_copy(y_sh.at[src_v.at[0]], rows0, g0)

        @pl.loop(0, chn, step=2)
        def _(j):
            a1 = pltpu.async_copy(y_sh.at[src_v.at[j + 1]], rows1, g1)
            pltpu.make_async_copy(y_sh.at[src_v.at[j]], rows0, g0).wait()
            pltpu.sync_copy(rows0, acc_sh.at[dst_v.at[j]], add=True)

            @pl.when(j + 2 < chn)
            def _():
                pltpu.async_copy(y_sh.at[src_v.at[j + 2]], rows0, g0)

            a1.wait()
            pltpu.sync_copy(rows1, acc_sh.at[dst_v.at[j + 1]], add=True)

        plsc.subcore_barrier()
        pltpu.sync_copy(acc_sh.at[pl.ds(s * RPT, RPT)],
                        out_hbm.at[c, pl.ds(s * RPT, RPT)])

    return k(y, ei3)


R = 2000


def _dinvF(degp0, degp1, pid, pad):
    deg = degp0 + degp1 + 1.0
    row0 = (lax.broadcasted_iota(jnp.int32, (R, 1), 0) == 0) & (pid == 0)
    deg = deg - jnp.where(row0, float(pad), 0.0)
    return lax.rsqrt(deg)


def _corr(pid, pad):
    row0 = (lax.broadcasted_iota(jnp.int32, (R, 1), 0) == 0) & (pid == 0)
    return jnp.where(row0, 1.0 - float(pad), 1.0)


def _tc1(x, W1, degp, pad):
    def body(x_ref, w_ref, degp_ref, y_ref):
        pid = pl.program_id(0)
        dinvF = _dinvF(degp_ref[0], degp_ref[1], pid, pad)
        xw = jnp.dot(x_ref[...], w_ref[...], preferred_element_type=jnp.float32)
        y_ref[...] = xw * jnp.dot(dinvF, jnp.full((L, HID), 1.0 / L, jnp.float32),
                                  preferred_element_type=jnp.float32)

    return pl.pallas_call(
        body,
        grid=(N // R,),
        in_specs=[
            pl.BlockSpec((R, 128), lambda i: (i, 0)),
            pl.BlockSpec((128, HID), lambda i: (0, 0)),
            pl.BlockSpec((NC, R, L), lambda i: (0, i, 0)),
        ],
        out_specs=pl.BlockSpec((R, HID), lambda i: (i, 0)),
        out_shape=jax.ShapeDtypeStruct((N, HID), jnp.float32),
    )(x, W1, degp)


def _tc2(a1p, y1, degp, b1, W2, pad):
    def body(a1p_ref, y1_ref, degp_ref, b1_ref, w2_ref, o_ref):
        pid = pl.program_id(0)
        dinvF = _dinvF(degp_ref[0], degp_ref[1], pid, pad)
        dinv64 = jnp.dot(dinvF, jnp.full((L, HID), 1.0 / L, jnp.float32),
                         preferred_element_type=jnp.float32)
        a1 = a1p_ref[0] + y1_ref[...] * _corr(pid, pad) + a1p_ref[1]
        h = jnp.maximum(dinv64 * a1 + b1_ref[...], 0.0)
        w2b = jnp.broadcast_to(w2_ref[...], (HID, L))
        o_ref[...] = dinvF * jnp.dot(h, w2b, preferred_element_type=jnp.float32)

    return pl.pallas_call(
        body,
        grid=(N // R,),
        in_specs=[
            pl.BlockSpec((NC, R, HID), lambda i: (0, i, 0)),
            pl.BlockSpec((R, HID), lambda i: (i, 0)),
            pl.BlockSpec((NC, R, L), lambda i: (0, i, 0)),
            pl.BlockSpec((HID,), lambda i: (0,)),
            pl.BlockSpec((HID, 1), lambda i: (0, 0)),
        ],
        out_specs=pl.BlockSpec((R, L), lambda i: (i, 0)),
        out_shape=jax.ShapeDtypeStruct((N, L), jnp.float32),
    )(a1p, y1, degp, b1, W2)


def _tc3(a2p, y2p, degp, b2, pad):
    def body(a2p_ref, y2p_ref, degp_ref, b2_ref, o_ref):
        pid = pl.program_id(0)
        dinvF = _dinvF(degp_ref[0], degp_ref[1], pid, pad)
        a2 = a2p_ref[0] + y2p_ref[...] * _corr(pid, pad) + a2p_ref[1]
        oF = jax.nn.sigmoid(dinvF * a2 + b2_ref[...])
        o_ref[...] = jnp.dot(oF, jnp.full((L, 1), 1.0 / L, jnp.float32),
                             preferred_element_type=jnp.float32)

    return pl.pallas_call(
        body,
        grid=(N // R,),
        in_specs=[
            pl.BlockSpec((NC, R, L), lambda i: (0, i, 0)),
            pl.BlockSpec((R, L), lambda i: (i, 0)),
            pl.BlockSpec((NC, R, L), lambda i: (0, i, 0)),
            pl.BlockSpec((1,), lambda i: (0,)),
        ],
        out_specs=pl.BlockSpec((R, 1), lambda i: (i, 0)),
        out_shape=jax.ShapeDtypeStruct((N, 1), jnp.float32),
    )(a2p, y2p, degp, b2)


def kernel(x, edge_index, W1, b1, W2, b2):
    e = edge_index.shape[1]
    grain = NW * CB * 4
    chn = 4 * (-(-e // grain))
    e_pad = chn * NW * CB
    pad = e_pad - e
    ei3 = jnp.pad(edge_index.astype(jnp.int32),
                  ((0, 0), (0, pad))).reshape(2, NW, chn, CB)

    degp = _sc_deg(ei3)
    y1 = _tc1(x, W1, degp, pad)
    a1p = _sc_agg(y1, ei3, HID)
    y2p = _tc2(a1p, y1, degp, b1, W2, pad)
    a2p = _sc_agg(y2p, ei3, L)
    return _tc3(a2p, y2p, degp, b2, pad)


# speedup vs baseline: 1.1703x; 1.0948x over previous
import functools

import jax
import jax.numpy as jnp
from jax import lax
from jax.experimental import pallas as pl
from jax.experimental.pallas import tpu as pltpu
from jax.experimental.pallas import tpu_sc as plsc

N = 10000
HID = 64
NC = 2
NS = 16
L = 16
NW = NC * NS
CB = 128
N_ACC = 10240
N_TC = 10240
RPT = N_ACC // NS


def _mesh():
    return plsc.VectorSubcoreMesh(core_axis_name="c", subcore_axis_name="s")


_SC_PARAMS = pltpu.CompilerParams(use_tc_tiling_on_sc=False)


def _fill(ref, rows, cols, val):
    @pl.loop(0, rows)
    def _(i):
        @pl.loop(0, cols, step=L)
        def _(j):
            ref.at[pl.ds(i, 1), pl.ds(j, L)][...] = jnp.full((1, L), val, jnp.float32)


def _sc_deg(ei3):
    chn = ei3.shape[2]

    @functools.partial(
        pl.kernel,
        out_type=jax.ShapeDtypeStruct((NC, N_ACC, L), jnp.float32),
        mesh=_mesh(),
        scratch_types=[
            pltpu.VMEM((chn, CB), jnp.int32),
            pltpu.VMEM((CB, L), jnp.float32),
            pltpu.VMEM((CB, L), jnp.float32),
            pltpu.VMEM_SHARED((N_ACC, L), jnp.float32),
            pltpu.SemaphoreType.DMA,
        ],
        compiler_params=_SC_PARAMS,
    )
    def k(ei_hbm, out_hbm, dst_v, ones_v, zb_v, acc_sh, sem):
        c = lax.axis_index("c")
        s = lax.axis_index("s")
        w = s * NC + c
        _fill(ones_v, CB, L, 1.0)
        _fill(zb_v, CB, L, 0.0)

        @pl.loop(0, RPT, step=CB)
        def _(r):
            pltpu.sync_copy(zb_v, acc_sh.at[pl.ds(s * RPT + r, CB)])

        plsc.subcore_barrier()
        pltpu.sync_copy(ei_hbm.at[1, w], dst_v)

        @pl.loop(0, chn, step=8)
        def _(j):
            for b in range(8):
                pltpu.async_copy(ones_v, acc_sh.at[dst_v.at[j + b]], sem, add=True)
            for b in range(8):
                pltpu.make_async_copy(ones_v, acc_sh.at[dst_v.at[j + b]], sem).wait()

        plsc.subcore_barrier()
        pltpu.sync_copy(acc_sh.at[pl.ds(s * RPT, RPT)],
                        out_hbm.at[c, pl.ds(s * RPT, RPT)])

    return k(ei3)


def _sc_agg(y, ei3, d):
    chn = ei3.shape[2]
    npt = N_TC // NS

    @functools.partial(
        pl.kernel,
        out_type=jax.ShapeDtypeStruct((NC, N_ACC, d), jnp.float32),
        mesh=_mesh(),
        scratch_types=[
            pltpu.VMEM((chn, CB), jnp.int32),
            pltpu.VMEM((chn, CB), jnp.int32),
            pltpu.VMEM((CB, d), jnp.float32),
            pltpu.VMEM((CB, d), jnp.float32),
            pltpu.VMEM((CB, d), jnp.float32),
            pltpu.VMEM_SHARED((N_TC, d), jnp.float32),
            pltpu.VMEM_SHARED((N_ACC, d), jnp.float32),
        ] + [pltpu.SemaphoreType.DMA] * 2,
        compiler_params=_SC_PARAMS,
    )
    def k(y_hbm, ei_hbm, out_hbm,
          src_v, dst_v, rows0, rows1, zb_v, y_sh, acc_sh, g0, g1):
        c = lax.axis_index("c")
        s = lax.axis_index("s")
        w = s * NC + c
        _fill(zb_v, CB, d, 0.0)

        @pl.loop(0, RPT, step=CB)
        def _(r):
            pltpu.sync_copy(zb_v, acc_sh.at[pl.ds(s * RPT + r, CB)])

        pltpu.sync_copy(y_hbm.at[pl.ds(s * npt, npt)], y_sh.at[pl.ds(s * npt, npt)])
        pltpu.sync_copy(ei_hbm.at[0, w], src_v)
        pltpu.sync_copy(ei_hbm.at[1, w], dst_v)
        plsc.subcore_barrier()

        pltpu.async_copy(y_sh.at[src_v.at[0]], rows0, g0)

        @pl.loop(0, chn, step=2)
        def _(j):
            a1 = pltpu.async_copy(y_sh.at[src_v.at[j + 1]], rows1, g1)
            pltpu.make_async_copy(y_sh.at[src_v.at[j]], rows0, g0).wait()
            pltpu.sync_copy(rows0, acc_sh.at[dst_v.at[j]], add=True)

            @pl.when(j + 2 < chn)
            def _():
                pltpu.async_copy(y_sh.at[src_v.at[j + 2]], rows0, g0)

            a1.wait()
            pltpu.sync_copy(rows1, acc_sh.at[dst_v.at[j + 1]], add=True)

        plsc.subcore_barrier()
        pltpu.sync_copy(acc_sh.at[pl.ds(s * RPT, RPT)],
                        out_hbm.at[c, pl.ds(s * RPT, RPT)])

    return k(y, ei3)


R = 2048
RQ = R * L // 128


def _iota2(shape, dim):
    return lax.broadcasted_iota(jnp.int32, shape, dim)


def _pad0(pid, shape, pad):
    sel = (_iota2(shape, 0) == 0)
    if shape[1] == 128:
        sel = sel & (_iota2(shape, 1) < L)
    return jnp.where(sel & (pid == 0), float(pad), 0.0)


def _dinv_nodes(degq_ref, pid, pad, width):
    degP = degq_ref[0] + degq_ref[1] + 1.0 - _pad0(pid, (RQ, 128), pad)
    S = (_iota2((128, 8), 0) == _iota2((128, 8), 1) * L).astype(jnp.float32)
    dinv8 = lax.rsqrt(jnp.dot(degP, S, preferred_element_type=jnp.float32))
    T = (_iota2((R, RQ), 1) == _iota2((R, RQ), 0) // 8).astype(jnp.float32)
    Z = jnp.dot(T, dinv8, preferred_element_type=jnp.float32)
    M = (_iota2((R, 8), 1) == _iota2((R, 8), 0) % 8).astype(jnp.float32)
    return jnp.dot(Z * M, jnp.ones((8, width), jnp.float32),
                   preferred_element_type=jnp.float32)


def _pack16(v16):
    vw = jnp.dot(v16, jnp.full((L, 128), 1.0 / L, jnp.float32),
                 preferred_element_type=jnp.float32)
    M128 = (_iota2((R, 128), 1) // L == _iota2((R, 128), 0) % 8).astype(jnp.float32)
    Tt = (_iota2((RQ, R), 0) == _iota2((RQ, R), 1) // 8).astype(jnp.float32)
    return jnp.dot(Tt, vw * M128, preferred_element_type=jnp.float32)


def _tc1(x, W1, degq, pad):
    def body(x_ref, w_ref, degq_ref, y_ref):
        pid = pl.program_id(0)
        dinv64 = _dinv_nodes(degq_ref, pid, pad, HID)
        xw = jnp.dot(x_ref[...], w_ref[...], preferred_element_type=jnp.float32)
        y_ref[...] = xw * dinv64

    return pl.pallas_call(
        body,
        grid=(N_TC // R,),
        in_specs=[
            pl.BlockSpec((R, 128), lambda i: (i, 0)),
            pl.BlockSpec((128, HID), lambda i: (0, 0)),
            pl.BlockSpec((NC, RQ, 128), lambda i: (0, i, 0)),
        ],
        out_specs=pl.BlockSpec((R, HID), lambda i: (i, 0)),
        out_shape=jax.ShapeDtypeStruct((N_TC, HID), jnp.float32),
    )(x, W1, degq)


def _tc2(a1p, y1, degq, b1, W2, pad):
    def body(a1p_ref, y1_ref, degq_ref, b1_ref, w2_ref, o_ref):
        pid = pl.program_id(0)
        dinvF = _dinv_nodes(degq_ref, pid, pad, L)
        dinv64 = jnp.dot(dinvF, jnp.full((L, HID), 1.0 / L, jnp.float32),
                         preferred_element_type=jnp.float32)
        corr = 1.0 - _pad0(pid, (R, 1), pad)
        a1 = a1p_ref[0] + y1_ref[...] * corr + a1p_ref[1]
        h = jnp.maximum(dinv64 * a1 + b1_ref[...], 0.0)
        w2b = jnp.broadcast_to(w2_ref[...], (HID, L))
        y2F = dinvF * jnp.dot(h, w2b, preferred_element_type=jnp.float32)
        o_ref[...] = _pack16(y2F)

    return pl.pallas_call(
        body,
        grid=(N_TC // R,),
        in_specs=[
            pl.BlockSpec((NC, R, HID), lambda i: (0, i, 0)),
            pl.BlockSpec((R, HID), lambda i: (i, 0)),
            pl.BlockSpec((NC, RQ, 128), lambda i: (0, i, 0)),
            pl.BlockSpec((HID,), lambda i: (0,)),
            pl.BlockSpec((HID, 1), lambda i: (0, 0)),
        ],
        out_specs=pl.BlockSpec((RQ, 128), lambda i: (i, 0)),
        out_shape=jax.ShapeDtypeStruct((N_TC * L // 128, 128), jnp.float32),
    )(a1p, y1, degq, b1, W2)


def _tc3(a2q, y2q, degq, b2, pad):
    def body(a2q_ref, y2q_ref, degq_ref, b2_ref, o_ref):
        pid = pl.program_id(0)
        degP = degq_ref[0] + degq_ref[1] + 1.0 - _pad0(pid, (RQ, 128), pad)
        dinvP = lax.rsqrt(degP)
        corrP = 1.0 - _pad0(pid, (RQ, 128), pad)
        a2 = a2q_ref[0] + y2q_ref[...] * corrP + a2q_ref[1]
        o_ref[...] = jax.nn.sigmoid(dinvP * a2 + b2_ref[...])

    return pl.pallas_call(
        body,
        grid=(N_TC // R,),
        in_specs=[
            pl.BlockSpec((NC, RQ, 128), lambda i: (0, i, 0)),
            pl.BlockSpec((RQ, 128), lambda i: (i, 0)),
            pl.BlockSpec((NC, RQ, 128), lambda i: (0, i, 0)),
            pl.BlockSpec((1,), lambda i: (0,)),
        ],
        out_specs=pl.BlockSpec((RQ, 128), lambda i: (i, 0)),
        out_shape=jax.ShapeDtypeStruct((N_TC * L // 128, 128), jnp.float32),
    )(a2q, y2q, degq, b2)


def kernel(x, edge_index, W1, b1, W2, b2):
    e = edge_index.shape[1]
    grain = NW * CB * 4
    chn = 4 * (-(-e // grain))
    e_pad = chn * NW * CB
    pad = e_pad - e
    ei3 = jnp.pad(edge_index.astype(jnp.int32),
                  ((0, 0), (0, pad))).reshape(2, NW, chn, CB)
    x_p = jnp.pad(x, ((0, N_TC - N), (0, 0)))

    degp = _sc_deg(ei3)
    degq = degp.reshape(NC, N_ACC * L // 128, 128)
    y1 = _tc1(x_p, W1, degq, pad)
    a1p = _sc_agg(y1, ei3, HID)
    y2q = _tc2(a1p, y1, degq, b1, W2, pad)
    y2p = y2q.reshape(N_TC, L)
    a2p = _sc_agg(y2p, ei3, L)
    a2q = a2p.reshape(NC, N_ACC * L // 128, 128)
    outq = _tc3(a2q, y2q, degq, b2, pad)
    return outq.reshape(N_TC, L)[:N, :1]


# speedup vs baseline: 1.2858x; 1.0987x over previous
import functools

import jax
import jax.numpy as jnp
from jax import lax
from jax.experimental import pallas as pl
from jax.experimental.pallas import tpu as pltpu
from jax.experimental.pallas import tpu_sc as plsc

N = 10000
HID = 64
NC = 2
NS = 16
L = 16
NW = NC * NS
CB = 128
N_ACC = 10240
N_TC = 10240
RPT = N_ACC // NS


def _mesh():
    return plsc.VectorSubcoreMesh(core_axis_name="c", subcore_axis_name="s")


_SC_PARAMS = pltpu.CompilerParams(use_tc_tiling_on_sc=False)


def _fill(ref, rows, cols, val):
    @pl.loop(0, rows)
    def _(i):
        @pl.loop(0, cols, step=L)
        def _(j):
            ref.at[pl.ds(i, 1), pl.ds(j, L)][...] = jnp.full((1, L), val, jnp.float32)


def _sc_deg(ei3):
    chn = ei3.shape[2]

    @functools.partial(
        pl.kernel,
        out_type=jax.ShapeDtypeStruct((NC, N_ACC, L), jnp.float32),
        mesh=_mesh(),
        scratch_types=[
            pltpu.VMEM((chn, CB), jnp.int32),
            pltpu.VMEM((CB, L), jnp.float32),
            pltpu.VMEM((CB, L), jnp.float32),
            pltpu.VMEM_SHARED((N_ACC, L), jnp.float32),
            pltpu.SemaphoreType.DMA,
        ],
        compiler_params=_SC_PARAMS,
    )
    def k(ei_hbm, out_hbm, dst_v, ones_v, zb_v, acc_sh, sem):
        c = lax.axis_index("c")
        s = lax.axis_index("s")
        w = s * NC + c
        _fill(ones_v, CB, L, 1.0)
        _fill(zb_v, CB, L, 0.0)

        @pl.loop(0, RPT, step=CB)
        def _(r):
            pltpu.sync_copy(zb_v, acc_sh.at[pl.ds(s * RPT + r, CB)])

        plsc.subcore_barrier()
        pltpu.sync_copy(ei_hbm.at[1, w], dst_v)

        @pl.loop(0, chn, step=8)
        def _(j):
            for b in range(8):
                pltpu.async_copy(ones_v, acc_sh.at[dst_v.at[j + b]], sem, add=True)
            for b in range(8):
                pltpu.make_async_copy(ones_v, acc_sh.at[dst_v.at[j + b]], sem).wait()

        plsc.subcore_barrier()
        pltpu.sync_copy(acc_sh.at[pl.ds(s * RPT, RPT)],
                        out_hbm.at[c, pl.ds(s * RPT, RPT)])

    return k(ei3)


def _sc_agg(y, ei3, d):
    chn = ei3.shape[2]
    npt = N_TC // NS

    @functools.partial(
        pl.kernel,
        out_type=jax.ShapeDtypeStruct((NC, N_ACC, d), jnp.float32),
        mesh=_mesh(),
        scratch_types=[
            pltpu.VMEM((chn, CB), jnp.int32),
            pltpu.VMEM((chn, CB), jnp.int32),
            pltpu.VMEM((CB, d), jnp.float32),
            pltpu.VMEM((CB, d), jnp.float32),
            pltpu.VMEM((CB, d), jnp.float32),
            pltpu.VMEM_SHARED((N_TC, d), jnp.float32),
            pltpu.VMEM_SHARED((N_ACC, d), jnp.float32),
        ] + [pltpu.SemaphoreType.DMA] * 2,
        compiler_params=_SC_PARAMS,
    )
    def k(y_hbm, ei_hbm, out_hbm,
          src_v, dst_v, rows0, rows1, zb_v, y_sh, acc_sh, g0, g1):
        c = lax.axis_index("c")
        s = lax.axis_index("s")
        w = s * NC + c
        _fill(zb_v, CB, d, 0.0)

        @pl.loop(0, RPT, step=CB)
        def _(r):
            pltpu.sync_copy(zb_v, acc_sh.at[pl.ds(s * RPT + r, CB)])

        pltpu.sync_copy(y_hbm.at[pl.ds(s * npt, npt)], y_sh.at[pl.ds(s * npt, npt)])
        pltpu.sync_copy(ei_hbm.at[0, w], src_v)
        pltpu.sync_copy(ei_hbm.at[1, w], dst_v)
        plsc.subcore_barrier()

        pltpu.async_copy(y_sh.at[src_v.at[0]], rows0, g0)

        @pl.loop(0, chn, step=2)
        def _(j):
            a1 = pltpu.async_copy(y_sh.at[src_v.at[j + 1]], rows1, g1)
            pltpu.make_async_copy(y_sh.at[src_v.at[j]], rows0, g0).wait()
            pltpu.sync_copy(rows0, acc_sh.at[dst_v.at[j]], add=True)

            @pl.when(j + 2 < chn)
            def _():
                pltpu.async_copy(y_sh.at[src_v.at[j + 2]], rows0, g0)

            a1.wait()
            pltpu.sync_copy(rows1, acc_sh.at[dst_v.at[j + 1]], add=True)

        plsc.subcore_barrier()
        pltpu.sync_copy(acc_sh.at[pl.ds(s * RPT, RPT)],
                        out_hbm.at[c, pl.ds(s * RPT, RPT)])

    return k(y, ei3)


R = 2048
RQ = R * L // 128
RH = 1024
RHQ = RH * L // 128
NH = N_TC // 2


def _iota2(shape, dim):
    return lax.broadcasted_iota(jnp.int32, shape, dim)


def _pad0(pid, shape, pad):
    sel = (_iota2(shape, 0) == 0)
    if shape[1] == 128:
        sel = sel & (_iota2(shape, 1) < L)
    return jnp.where(sel & (pid == 0), float(pad), 0.0)


def _dinv_nodes(degq_ref, pid, pad, width, rows):
    rq = rows * L // 128
    degP = degq_ref[0] + degq_ref[1] + 1.0 - _pad0(pid, (rq, 128), pad)
    S = (_iota2((128, 8), 0) == _iota2((128, 8), 1) * L).astype(jnp.float32)
    dinv8 = lax.rsqrt(jnp.dot(degP, S, preferred_element_type=jnp.float32))
    T = (_iota2((rows, rq), 1) == _iota2((rows, rq), 0) // 8).astype(jnp.float32)
    Z = jnp.dot(T, dinv8, preferred_element_type=jnp.float32)
    M = (_iota2((rows, 8), 1) == _iota2((rows, 8), 0) % 8).astype(jnp.float32)
    return jnp.dot(Z * M, jnp.ones((8, width), jnp.float32),
                   preferred_element_type=jnp.float32)


def _pack16(v16, rows):
    vw = jnp.dot(v16, jnp.full((L, 128), 1.0 / L, jnp.float32),
                 preferred_element_type=jnp.float32)
    M128 = (_iota2((rows, 128), 1) // L == _iota2((rows, 128), 0) % 8).astype(jnp.float32)
    Tt = (_iota2((rows * L // 128, rows), 0) == _iota2((rows * L // 128, rows), 1) // 8).astype(jnp.float32)
    return jnp.dot(Tt, vw * M128, preferred_element_type=jnp.float32)


def _tc1(x, W1, degq, pad):
    def body(xa_ref, xb_ref, w_ref, dqa_ref, dqb_ref, y_ref):
        pid = pl.program_id(0)
        da = _dinv_nodes(dqa_ref, pid, pad, HID, RH)
        db = _dinv_nodes(dqb_ref, pid, 0.0, HID, RH)
        ya = jnp.dot(xa_ref[...], w_ref[...], preferred_element_type=jnp.float32) * da
        yb = jnp.dot(xb_ref[...], w_ref[...], preferred_element_type=jnp.float32) * db
        y_ref[...] = jnp.concatenate([ya, yb], axis=1)

    nb = NH // RH
    return pl.pallas_call(
        body,
        grid=(nb,),
        in_specs=[
            pl.BlockSpec((RH, 128), lambda i: (i, 0)),
            pl.BlockSpec((RH, 128), lambda i, nb=nb: (i + nb, 0)),
            pl.BlockSpec((128, HID), lambda i: (0, 0)),
            pl.BlockSpec((NC, RHQ, 128), lambda i: (0, i, 0)),
            pl.BlockSpec((NC, RHQ, 128), lambda i, nb=nb: (0, i + nb, 0)),
        ],
        out_specs=pl.BlockSpec((RH, 128), lambda i: (i, 0)),
        out_shape=jax.ShapeDtypeStruct((NH, 128), jnp.float32),
    )(x, x, W1, degq, degq)


def _tc2(a1p, y1, degq, b1, W2, pad):
    a1q = a1p.reshape(NC, NH, 128)

    def body(a1q_ref, y1_ref, dqa_ref, dqb_ref, b1_ref, w2_ref, o_ref):
        pid = pl.program_id(0)
        da = _dinv_nodes(dqa_ref, pid, pad, HID, RH)
        db = _dinv_nodes(dqb_ref, pid, 0.0, HID, RH)
        corr = 1.0 - _pad0(pid, (RH, 1), pad)
        B = a1q_ref[0] + a1q_ref[1]
        w2b = jnp.broadcast_to(w2_ref[...], (HID, L))
        ha = jnp.maximum(da * (B[:, :HID] + y1_ref[:, :HID] * corr) + b1_ref[...], 0.0)
        hb = jnp.maximum(db * (B[:, HID:] + y1_ref[:, HID:]) + b1_ref[...], 0.0)
        y2a = da[:, :L] * jnp.dot(ha, w2b, preferred_element_type=jnp.float32)
        y2b = db[:, :L] * jnp.dot(hb, w2b, preferred_element_type=jnp.float32)
        o_ref[...] = jnp.stack([_pack16(y2a, RH), _pack16(y2b, RH)])

    nb = NH // RH
    return pl.pallas_call(
        body,
        grid=(nb,),
        in_specs=[
            pl.BlockSpec((NC, RH, 128), lambda i: (0, i, 0)),
            pl.BlockSpec((RH, 128), lambda i: (i, 0)),
            pl.BlockSpec((NC, RHQ, 128), lambda i: (0, i, 0)),
            pl.BlockSpec((NC, RHQ, 128), lambda i, nb=nb: (0, i + nb, 0)),
            pl.BlockSpec((HID,), lambda i: (0,)),
            pl.BlockSpec((HID, 1), lambda i: (0, 0)),
        ],
        out_specs=pl.BlockSpec((2, RHQ, 128), lambda i: (0, i, 0)),
        out_shape=jax.ShapeDtypeStruct((2, NH * L // 128, 128), jnp.float32),
    )(a1q, y1, degq, degq, b1, W2)


def _tc3(a2q, y2q, degq, b2, pad):
    def body(a2q_ref, y2q_ref, degq_ref, b2_ref, o_ref):
        pid = pl.program_id(0)
        degP = degq_ref[0] + degq_ref[1] + 1.0 - _pad0(pid, (RQ, 128), pad)
        dinvP = lax.rsqrt(degP)
        corrP = 1.0 - _pad0(pid, (RQ, 128), pad)
        a2 = a2q_ref[0] + y2q_ref[...] * corrP + a2q_ref[1]
        o_ref[...] = jax.nn.sigmoid(dinvP * a2 + b2_ref[...])

    return pl.pallas_call(
        body,
        grid=(N_TC // R,),
        in_specs=[
            pl.BlockSpec((NC, RQ, 128), lambda i: (0, i, 0)),
            pl.BlockSpec((RQ, 128), lambda i: (i, 0)),
            pl.BlockSpec((NC, RQ, 128), lambda i: (0, i, 0)),
            pl.BlockSpec((1,), lambda i: (0,)),
        ],
        out_specs=pl.BlockSpec((RQ, 128), lambda i: (i, 0)),
        out_shape=jax.ShapeDtypeStruct((N_TC * L // 128, 128), jnp.float32),
    )(a2q, y2q, degq, b2)


def kernel(x, edge_index, W1, b1, W2, b2):
    e = edge_index.shape[1]
    grain = NW * CB * 4
    chn = 4 * (-(-e // grain))
    e_pad = chn * NW * CB
    pad = e_pad - e
    ei = jnp.pad(edge_index.astype(jnp.int32), ((0, 0), (0, pad)))
    ei3 = ei.reshape(2, NW, chn, CB)
    eip3 = jnp.where(ei < NH, 2 * ei, 2 * (ei - NH) + 1).reshape(2, NW, chn, CB)
    x_p = jnp.pad(x, ((0, N_TC - N), (0, 0)))

    degp = _sc_deg(ei3)
    degq = degp.reshape(NC, N_ACC * L // 128, 128)
    y1 = _tc1(x_p, W1, degq, pad)
    a1p = _sc_agg(y1.reshape(N_TC, HID), eip3, HID)
    y2q = _tc2(a1p, y1, degq, b1, W2, pad)
    y2p = y2q.reshape(N_TC, L)
    a2p = _sc_agg(y2p, ei3, L)
    a2q = a2p.reshape(NC, N_ACC * L // 128, 128)
    outq = _tc3(a2q, y2q.reshape(N_TC * L // 128, 128), degq, b2, pad)
    return outq.reshape(N_TC, L)[:N, :1]


# speedup vs baseline: 1.3273x; 1.0323x over previous
import functools

import jax
import jax.numpy as jnp
from jax import lax
from jax.experimental import pallas as pl
from jax.experimental.pallas import tpu as pltpu
from jax.experimental.pallas import tpu_sc as plsc

N = 10000
HID = 64
NC = 2
NS = 16
L = 16
NW = NC * NS
CB = 128
N_ACC = 10240
N_TC = 10240
RPT = N_ACC // NS


def _mesh():
    return plsc.VectorSubcoreMesh(core_axis_name="c", subcore_axis_name="s")


_SC_PARAMS = pltpu.CompilerParams(use_tc_tiling_on_sc=False)


def _fill(ref, rows, cols, val):
    @pl.loop(0, rows)
    def _(i):
        @pl.loop(0, cols, step=L)
        def _(j):
            ref.at[pl.ds(i, 1), pl.ds(j, L)][...] = jnp.full((1, L), val, jnp.float32)


def _sc_deg(ei3):
    chn = ei3.shape[2]

    @functools.partial(
        pl.kernel,
        out_type=jax.ShapeDtypeStruct((NC, N_ACC, L), jnp.float32),
        mesh=_mesh(),
        scratch_types=[
            pltpu.VMEM((chn, CB), jnp.int32),
            pltpu.VMEM((CB, L), jnp.float32),
            pltpu.VMEM((CB, L), jnp.float32),
            pltpu.VMEM_SHARED((N_ACC, L), jnp.float32),
            pltpu.SemaphoreType.DMA,
        ],
        compiler_params=_SC_PARAMS,
    )
    def k(ei_hbm, out_hbm, dst_v, ones_v, zb_v, acc_sh, sem):
        c = lax.axis_index("c")
        s = lax.axis_index("s")
        w = s * NC + c
        _fill(ones_v, CB, L, 1.0)
        _fill(zb_v, CB, L, 0.0)

        @pl.loop(0, RPT, step=CB)
        def _(r):
            pltpu.sync_copy(zb_v, acc_sh.at[pl.ds(s * RPT + r, CB)])

        plsc.subcore_barrier()
        pltpu.sync_copy(ei_hbm.at[1, w], dst_v)

        @pl.loop(0, chn, step=8)
        def _(j):
            for b in range(8):
                pltpu.async_copy(ones_v, acc_sh.at[dst_v.at[j + b]], sem, add=True)
            for b in range(8):
                pltpu.make_async_copy(ones_v, acc_sh.at[dst_v.at[j + b]], sem).wait()

        plsc.subcore_barrier()
        pltpu.sync_copy(acc_sh.at[pl.ds(s * RPT, RPT)],
                        out_hbm.at[c, pl.ds(s * RPT, RPT)])

    return k(ei3)


def _sc_agg(y, ei3, d):
    chn = ei3.shape[2]
    npt = N_TC // NS

    @functools.partial(
        pl.kernel,
        out_type=jax.ShapeDtypeStruct((NC, N_ACC, d), jnp.float32),
        mesh=_mesh(),
        scratch_types=[
            pltpu.VMEM((chn, CB), jnp.int32),
            pltpu.VMEM((chn, CB), jnp.int32),
            pltpu.VMEM((CB, d), jnp.float32),
            pltpu.VMEM((CB, d), jnp.float32),
            pltpu.VMEM((CB, d), jnp.float32),
            pltpu.VMEM_SHARED((N_TC, d), jnp.float32),
            pltpu.VMEM_SHARED((N_ACC, d), jnp.float32),
        ] + [pltpu.SemaphoreType.DMA] * 2,
        compiler_params=_SC_PARAMS,
    )
    def k(y_hbm, ei_hbm, out_hbm,
          src_v, dst_v, rows0, rows1, zb_v, y_sh, acc_sh, g0, g1):
        c = lax.axis_index("c")
        s = lax.axis_index("s")
        w = s * NC + c
        _fill(zb_v, CB, d, 0.0)
        st = pltpu.async_copy(y_hbm.at[pl.ds(s * npt, npt)],
                              y_sh.at[pl.ds(s * npt, npt)], g0)
        i0 = pltpu.async_copy(ei_hbm.at[0, w], src_v, g1)
        i1 = pltpu.async_copy(ei_hbm.at[1, w], dst_v, g1)

        @pl.loop(0, RPT, step=CB)
        def _(r):
            pltpu.sync_copy(zb_v, acc_sh.at[pl.ds(s * RPT + r, CB)])

        st.wait()
        i0.wait()
        i1.wait()
        plsc.subcore_barrier()

        pltpu.async_copy(y_sh.at[src_v.at[0]], rows0, g0)

        @pl.loop(0, chn, step=2)
        def _(j):
            a1 = pltpu.async_copy(y_sh.at[src_v.at[j + 1]], rows1, g1)
            pltpu.make_async_copy(y_sh.at[src_v.at[j]], rows0, g0).wait()
            pltpu.sync_copy(rows0, acc_sh.at[dst_v.at[j]], add=True)

            @pl.when(j + 2 < chn)
            def _():
                pltpu.async_copy(y_sh.at[src_v.at[j + 2]], rows0, g0)

            a1.wait()
            pltpu.sync_copy(rows1, acc_sh.at[dst_v.at[j + 1]], add=True)

        plsc.subcore_barrier()
        pltpu.sync_copy(acc_sh.at[pl.ds(s * RPT, RPT)],
                        out_hbm.at[c, pl.ds(s * RPT, RPT)])

    return k(y, ei3)


R = 2048
RQ = R * L // 128
RH = 1024
RHQ = RH * L // 128
NH = N_TC // 2


def _iota2(shape, dim):
    return lax.broadcasted_iota(jnp.int32, shape, dim)


def _pad0(pid, shape, pad):
    sel = (_iota2(shape, 0) == 0)
    if shape[1] == 128:
        sel = sel & (_iota2(shape, 1) < L)
    return jnp.where(sel & (pid == 0), float(pad), 0.0)


def _dinv_nodes(degq_ref, pid, pad, width, rows):
    rq = rows * L // 128
    degP = degq_ref[0] + degq_ref[1] + 1.0 - _pad0(pid, (rq, 128), pad)
    S = (_iota2((128, 8), 0) == _iota2((128, 8), 1) * L).astype(jnp.float32)
    dinv8 = lax.rsqrt(jnp.dot(degP, S, preferred_element_type=jnp.float32))
    T = (_iota2((rows, rq), 1) == _iota2((rows, rq), 0) // 8).astype(jnp.float32)
    Z = jnp.dot(T, dinv8, preferred_element_type=jnp.float32)
    M = (_iota2((rows, 8), 1) == _iota2((rows, 8), 0) % 8).astype(jnp.float32)
    return jnp.dot(Z * M, jnp.ones((8, width), jnp.float32),
                   preferred_element_type=jnp.float32)


def _pack16(v16, rows):
    vw = jnp.dot(v16, jnp.full((L, 128), 1.0 / L, jnp.float32),
                 preferred_element_type=jnp.float32)
    M128 = (_iota2((rows, 128), 1) // L == _iota2((rows, 128), 0) % 8).astype(jnp.float32)
    Tt = (_iota2((rows * L // 128, rows), 0) == _iota2((rows * L // 128, rows), 1) // 8).astype(jnp.float32)
    return jnp.dot(Tt, vw * M128, preferred_element_type=jnp.float32)


def _tc1(x, W1, degq, pad):
    def body(xa_ref, xb_ref, w_ref, dqa_ref, dqb_ref, y_ref):
        pid = pl.program_id(0)
        da = _dinv_nodes(dqa_ref, pid, pad, HID, RH)
        db = _dinv_nodes(dqb_ref, pid, 0.0, HID, RH)
        ya = jnp.dot(xa_ref[...], w_ref[...], preferred_element_type=jnp.float32) * da
        yb = jnp.dot(xb_ref[...], w_ref[...], preferred_element_type=jnp.float32) * db
        y_ref[...] = jnp.concatenate([ya, yb], axis=1)

    nb = NH // RH
    return pl.pallas_call(
        body,
        grid=(nb,),
        in_specs=[
            pl.BlockSpec((RH, 128), lambda i: (i, 0)),
            pl.BlockSpec((RH, 128), lambda i, nb=nb: (i + nb, 0)),
            pl.BlockSpec((128, HID), lambda i: (0, 0)),
            pl.BlockSpec((NC, RHQ, 128), lambda i: (0, i, 0)),
            pl.BlockSpec((NC, RHQ, 128), lambda i, nb=nb: (0, i + nb, 0)),
        ],
        out_specs=pl.BlockSpec((RH, 128), lambda i: (i, 0)),
        out_shape=jax.ShapeDtypeStruct((NH, 128), jnp.float32),
    )(x, x, W1, degq, degq)


def _tc2(a1p, y1, degq, b1, W2, pad):
    a1q = a1p.reshape(NC, NH, 128)

    def body(a1q_ref, y1_ref, dqa_ref, dqb_ref, b1_ref, w2_ref, o_ref):
        pid = pl.program_id(0)
        da = _dinv_nodes(dqa_ref, pid, pad, HID, RH)
        db = _dinv_nodes(dqb_ref, pid, 0.0, HID, RH)
        corr = 1.0 - _pad0(pid, (RH, 1), pad)
        B = a1q_ref[0] + a1q_ref[1]
        w2b = jnp.broadcast_to(w2_ref[...], (HID, L))
        ha = jnp.maximum(da * (B[:, :HID] + y1_ref[:, :HID] * corr) + b1_ref[...], 0.0)
        hb = jnp.maximum(db * (B[:, HID:] + y1_ref[:, HID:]) + b1_ref[...], 0.0)
        y2a = da[:, :L] * jnp.dot(ha, w2b, preferred_element_type=jnp.float32)
        y2b = db[:, :L] * jnp.dot(hb, w2b, preferred_element_type=jnp.float32)
        o_ref[...] = jnp.stack([_pack16(y2a, RH), _pack16(y2b, RH)])

    nb = NH // RH
    return pl.pallas_call(
        body,
        grid=(nb,),
        in_specs=[
            pl.BlockSpec((NC, RH, 128), lambda i: (0, i, 0)),
            pl.BlockSpec((RH, 128), lambda i: (i, 0)),
            pl.BlockSpec((NC, RHQ, 128), lambda i: (0, i, 0)),
            pl.BlockSpec((NC, RHQ, 128), lambda i, nb=nb: (0, i + nb, 0)),
            pl.BlockSpec((HID,), lambda i: (0,)),
            pl.BlockSpec((HID, 1), lambda i: (0, 0)),
        ],
        out_specs=pl.BlockSpec((2, RHQ, 128), lambda i: (0, i, 0)),
        out_shape=jax.ShapeDtypeStruct((2, NH * L // 128, 128), jnp.float32),
    )(a1q, y1, degq, degq, b1, W2)


def _tc3(a2q, y2q, degq, b2, pad):
    def body(a2q_ref, y2q_ref, degq_ref, b2_ref, o_ref):
        pid = pl.program_id(0)
        degP = degq_ref[0] + degq_ref[1] + 1.0 - _pad0(pid, (RQ, 128), pad)
        dinvP = lax.rsqrt(degP)
        corrP = 1.0 - _pad0(pid, (RQ, 128), pad)
        a2 = a2q_ref[0] + y2q_ref[...] * corrP + a2q_ref[1]
        o_ref[...] = jax.nn.sigmoid(dinvP * a2 + b2_ref[...])

    return pl.pallas_call(
        body,
        grid=(N_TC // R,),
        in_specs=[
            pl.BlockSpec((NC, RQ, 128), lambda i: (0, i, 0)),
            pl.BlockSpec((RQ, 128), lambda i: (i, 0)),
            pl.BlockSpec((NC, RQ, 128), lambda i: (0, i, 0)),
            pl.BlockSpec((1,), lambda i: (0,)),
        ],
        out_specs=pl.BlockSpec((RQ, 128), lambda i: (i, 0)),
        out_shape=jax.ShapeDtypeStruct((N_TC * L // 128, 128), jnp.float32),
    )(a2q, y2q, degq, b2)


def kernel(x, edge_index, W1, b1, W2, b2):
    e = edge_index.shape[1]
    grain = NW * CB * 4
    chn = 4 * (-(-e // grain))
    e_pad = chn * NW * CB
    pad = e_pad - e
    ei = jnp.pad(edge_index.astype(jnp.int32), ((0, 0), (0, pad)))
    ei3 = ei.reshape(2, NW, chn, CB)
    eip3 = jnp.where(ei < NH, 2 * ei, 2 * (ei - NH) + 1).reshape(2, NW, chn, CB)
    x_p = jnp.pad(x, ((0, N_TC - N), (0, 0)))

    degp = _sc_deg(ei3)
    degq = degp.reshape(NC, N_ACC * L // 128, 128)
    y1 = _tc1(x_p, W1, degq, pad)
    a1p = _sc_agg(y1.reshape(N_TC, HID), eip3, HID)
    y2q = _tc2(a1p, y1, degq, b1, W2, pad)
    y2p = y2q.reshape(N_TC, L)
    a2p = _sc_agg(y2p, ei3, L)
    a2q = a2p.reshape(NC, N_ACC * L // 128, 128)
    outq = _tc3(a2q, y2q.reshape(N_TC * L // 128, 128), degq, b2, pad)
    return outq.reshape(N_TC, L)[:N, :1]
